# baseline (speedup 1.0000x reference)
import functools

import jax
import jax.numpy as jnp
from jax import lax
from jax.experimental import pallas as pl
from jax.experimental.pallas import tpu as pltpu
from jax.experimental.pallas import tpu_sc as plsc

N_TOKENS = 32768
N_EMBD = 4096
N_CHOICES = 64
HIDDEN = N_EMBD // 2

BT = 256
C = 2
TPC = N_TOKENS // C

NW = 32
TPW = TPC // NW
LANES = 16


def _reduce_body(k_ref, x_ref, w1_ref, b1_ref, w2_ref, b2_ref, o_ref):
    h = jnp.dot(x_ref[...], w1_ref[...], preferred_element_type=jnp.float32)
    h = jnp.maximum(h + b1_ref[...], 0.0)
    logits = jnp.dot(h, w2_ref[...], preferred_element_type=jnp.float32)
    logits = logits + b2_ref[...]

    col = jax.lax.broadcasted_iota(jnp.int32, logits.shape, 1)
    b = jax.lax.bitcast_convert_type(logits, jnp.int32)
    key = b ^ ((b >> 31) & jnp.int32(0x7FFFFFFF))
    key = (key & jnp.int32(~63)) | (jnp.int32(63) - col)
    k1 = jnp.max(key, axis=-1, keepdims=True)
    k2 = jnp.max(
        jnp.where(key == k1, jnp.int32(-0x80000000), key), axis=-1, keepdims=True
    )
    i1 = jnp.int32(63) - (k1 & jnp.int32(63))
    i2 = jnp.int32(63) - (k2 & jnp.int32(63))

    def _unkey(kk):
        ub = kk & jnp.int32(~63)
        return jax.lax.bitcast_convert_type(
            ub ^ ((ub >> 31) & jnp.int32(0x7FFFFFFF)), jnp.float32
        )

    p1 = jax.nn.sigmoid(_unkey(k1) - _unkey(k2))
    k_is_1 = k_ref[0] == 1
    v1 = jnp.where(k_is_1, jnp.float32(1.0), p1)
    v2 = jnp.where(k_is_1, jnp.float32(0.0), 1.0 - p1)
    v1b = jax.lax.bitcast_convert_type(v1, jnp.int32)
    v2b = jax.lax.bitcast_convert_type(v2, jnp.int32)

    col8 = jax.lax.broadcasted_iota(jnp.int32, o_ref.shape, 1)
    o_ref[...] = jnp.where(
        col8 == 0,
        i1,
        jnp.where(col8 == 1, i2, jnp.where(col8 == 2, v1b, v2b)),
    )


def _router_reduce(x, W1, b1, W2, b2, k, chunk):
    base = chunk * (TPC // BT)
    return pl.pallas_call(
        _reduce_body,
        grid=(TPC // BT,),
        in_specs=[
            pl.BlockSpec(memory_space=pltpu.SMEM),
            pl.BlockSpec((BT, N_EMBD), lambda i: (i + base, 0)),
            pl.BlockSpec((N_EMBD, HIDDEN), lambda i: (0, 0)),
            pl.BlockSpec((1, HIDDEN), lambda i: (0, 0)),
            pl.BlockSpec((HIDDEN, N_CHOICES), lambda i: (0, 0)),
            pl.BlockSpec((1, N_CHOICES), lambda i: (0, 0)),
        ],
        out_specs=pl.BlockSpec((BT, 8), lambda i: (i, 0)),
        out_shape=jax.ShapeDtypeStruct((TPC, 8), jnp.int32),
    )(k, x, W1, b1, W2, b2)


def _scatter_body(pk_hbm, out_hbm, pkv, buf):
    wid = lax.axis_index("s") * 2 + lax.axis_index("c")
    base = wid * TPW

    pltpu.sync_copy(pk_hbm.at[pl.ds(base, TPW)], pkv)

    zero16 = jnp.zeros((LANES,), jnp.float32)

    def _zero_block(i, carry):
        for j in range(16):
            buf[pl.ds(i * (16 * LANES) + j * LANES, LANES)] = zero16
        return carry

    lax.fori_loop(0, TPW * N_CHOICES // (16 * LANES), _zero_block, 0)

    lane = lax.iota(jnp.int32, LANES)
    for g in range(TPW // LANES):
        rows = lane + g * LANES
        i1g = plsc.load_gather(pkv, [rows, jnp.zeros((LANES,), jnp.int32)])
        i2g = plsc.load_gather(pkv, [rows, jnp.ones((LANES,), jnp.int32)])
        v1g = plsc.bitcast(
            plsc.load_gather(pkv, [rows, jnp.full((LANES,), 2, jnp.int32)]),
            jnp.float32,
        )
        v2g = plsc.bitcast(
            plsc.load_gather(pkv, [rows, jnp.full((LANES,), 3, jnp.int32)]),
            jnp.float32,
        )
        flat = rows * N_CHOICES
        plsc.store_scatter(buf, [flat + i1g], v1g)
        plsc.store_scatter(buf, [flat + i2g], v2g)

    pltpu.sync_copy(buf, out_hbm.at[pl.ds(base * N_CHOICES, TPW * N_CHOICES)])


_scatter_sc = functools.partial(
    pl.kernel,
    out_type=jax.ShapeDtypeStruct((TPC * N_CHOICES,), jnp.float32),
    mesh=plsc.VectorSubcoreMesh(core_axis_name="c", subcore_axis_name="s"),
    compiler_params=pltpu.CompilerParams(needs_layout_passes=False),
    scratch_types=[
        pltpu.VMEM((TPW, 8), jnp.int32),
        pltpu.VMEM((TPW * N_CHOICES,), jnp.float32),
    ],
)(_scatter_body)


def kernel(x, W1, b1, W2, b2, k, training):
    k_arr = jnp.asarray(k, jnp.int32).reshape((1,))
    b1r = b1.reshape(1, HIDDEN)
    b2r = b2.reshape(1, N_CHOICES)
    chunks = []
    for c in range(C):
        pk = _router_reduce(x, W1, b1r, W2, b2r, k_arr, c)
        chunks.append(_scatter_sc(pk).reshape(TPC, N_CHOICES))
    return jnp.concatenate(chunks, axis=0)


# speedup vs baseline: 1.0279x; 1.0279x over previous
import functools

import jax
import jax.numpy as jnp
from jax import lax
from jax.experimental import pallas as pl
from jax.experimental.pallas import tpu as pltpu
from jax.experimental.pallas import tpu_sc as plsc

N_TOKENS = 32768
N_EMBD = 4096
N_CHOICES = 64
HIDDEN = N_EMBD // 2

BT = 256
NW = 32
TPW = N_TOKENS // NW
HALF = TPW // 2
LANES = 16

_IMIN = -0x80000000


def _logits_body(x_ref, w1_ref, b1_ref, w2_ref, b2_ref, o_ref):
    h = jnp.dot(x_ref[...], w1_ref[...], preferred_element_type=jnp.float32)
    h = jnp.maximum(h + b1_ref[...], 0.0)
    logits = jnp.dot(h, w2_ref[...], preferred_element_type=jnp.float32)
    o_ref[...] = logits + b2_ref[...]


@jax.jit
def _router_logits(x, W1, b1, W2, b2):
    return pl.pallas_call(
        _logits_body,
        grid=(N_TOKENS // BT,),
        in_specs=[
            pl.BlockSpec((BT, N_EMBD), lambda i: (i, 0)),
            pl.BlockSpec((N_EMBD, HIDDEN), lambda i: (0, 0)),
            pl.BlockSpec((1, HIDDEN), lambda i: (0, 0)),
            pl.BlockSpec((HIDDEN, N_CHOICES), lambda i: (0, 0)),
            pl.BlockSpec((1, N_CHOICES), lambda i: (0, 0)),
        ],
        out_specs=pl.BlockSpec((BT, N_CHOICES), lambda i: (i, 0)),
        out_shape=jax.ShapeDtypeStruct((N_TOKENS, N_CHOICES), jnp.float32),
    )(x, W1, b1, W2, b2)


def _topk_body(logits_hbm, k_hbm, out_hbm, lv, kv, buf):
    wid = lax.axis_index("s") * 2 + lax.axis_index("c")
    base = wid * TPW

    pltpu.sync_copy(k_hbm, kv)
    k_is_1 = kv[...] == 1

    zero16 = jnp.zeros((LANES,), jnp.float32)
    lane = lax.iota(jnp.int32, LANES)
    mask63 = jnp.int32(~63)
    m31 = jnp.int32(0x7FFFFFFF)

    def _unkey(kk):
        ub = kk & mask63
        return plsc.bitcast(ub ^ ((ub >> 31) & m31), jnp.float32)

    for h in range(TPW // HALF):
        pltpu.sync_copy(
            logits_hbm.at[pl.ds((base + h * HALF) * N_CHOICES, HALF * N_CHOICES)],
            lv,
        )

        def _group(g, carry, h=h):
            rows = g * LANES + lane
            m1 = jnp.full((LANES,), _IMIN, jnp.int32)
            m2 = jnp.full((LANES,), _IMIN, jnp.int32)
            obase = (h * HALF + g * LANES) * N_CHOICES
            for c in range(N_CHOICES):
                lg = plsc.load_gather(lv, [rows * N_CHOICES + jnp.int32(c)])
                b = plsc.bitcast(lg, jnp.int32)
                key = b ^ ((b >> 31) & m31)
                key = (key & mask63) | jnp.int32(63 - c)
                lo = jnp.minimum(m1, key)
                m1 = jnp.maximum(m1, key)
                m2 = jnp.maximum(m2, lo)
                buf[pl.ds(obase + c * LANES, LANES)] = zero16
            i1 = jnp.int32(63) - (m1 & jnp.int32(63))
            i2 = jnp.int32(63) - (m2 & jnp.int32(63))
            p1 = 1.0 / (1.0 + jnp.exp(_unkey(m2) - _unkey(m1)))
            v1 = jnp.where(k_is_1, jnp.float32(1.0), p1)
            v2 = jnp.where(k_is_1, jnp.float32(0.0), 1.0 - p1)
            flat = (h * HALF + g * LANES + lane) * N_CHOICES
            plsc.store_scatter(buf, [flat + i1], v1)
            plsc.store_scatter(buf, [flat + i2], v2)
            return carry

        lax.fori_loop(0, HALF // LANES, _group, 0)

    pltpu.sync_copy(buf, out_hbm.at[pl.ds(base * N_CHOICES, TPW * N_CHOICES)])


_topk_sc = functools.partial(
    pl.kernel,
    out_type=jax.ShapeDtypeStruct((N_TOKENS * N_CHOICES,), jnp.float32),
    mesh=plsc.VectorSubcoreMesh(core_axis_name="c", subcore_axis_name="s"),
    compiler_params=pltpu.CompilerParams(needs_layout_passes=False),
    scratch_types=[
        pltpu.VMEM((HALF * N_CHOICES,), jnp.float32),
        pltpu.VMEM((LANES,), jnp.int32),
        pltpu.VMEM((TPW * N_CHOICES,), jnp.float32),
    ],
)(_topk_body)


def kernel(x, W1, b1, W2, b2, k, training):
    logits = _router_logits(
        x, W1, b1.reshape(1, HIDDEN), W2, b2.reshape(1, N_CHOICES)
    )
    k_arr = jnp.full((LANES,), jnp.asarray(k, jnp.int32))
    flat = _topk_sc(logits.reshape(N_TOKENS * N_CHOICES), k_arr)
    return flat.reshape(N_TOKENS, N_CHOICES)
